# Initial kernel scaffold; baseline (speedup 1.0000x reference)
import jax
import jax.numpy as jnp
from jax.experimental import pallas as pl


def kernel(feature_value, tables):
    raise NotImplementedError("write your pallas kernel here")


# speedup vs baseline: 1.1644x; 1.1644x over previous
import functools

import jax
import jax.numpy as jnp
from jax import lax
from jax.experimental import pallas as pl
from jax.experimental.pallas import tpu as pltpu
from jax.experimental.pallas import tpu_sc as plsc


def _make_emb_kernel(F, V, D, B):
    info = plsc.get_sparse_core_info()
    NC, NS = info.num_cores, info.num_subcores
    NW = NC * NS
    assert B % NW == 0
    BPW = B // NW

    mesh = plsc.VectorSubcoreMesh(core_axis_name="c", subcore_axis_name="s")

    @functools.partial(
        pl.kernel,
        mesh=mesh,
        out_type=jax.ShapeDtypeStruct((F, B, D), jnp.float32),
        scratch_types=[
            pltpu.VMEM((BPW,), jnp.int32),
            pltpu.VMEM((BPW, D), jnp.float32),
            pltpu.SemaphoreType.DMA,
        ],
        compiler_params=pltpu.CompilerParams(use_tc_tiling_on_sc=False),
    )
    def emb(fv_hbm, tab_hbm, out_hbm, idx_v, rows_v, sem):
        wid = lax.axis_index("s") * NC + lax.axis_index("c")
        base = wid * BPW
        for i in range(F):
            pltpu.sync_copy(fv_hbm.at[i, pl.ds(base, BPW)], idx_v)
            pltpu.async_copy(tab_hbm.at[i].at[idx_v], rows_v, sem).wait()
            pltpu.sync_copy(rows_v, out_hbm.at[i, pl.ds(base, BPW)])

    return emb


def kernel(feature_value, tables):
    F, V, D = tables.shape
    B = feature_value.shape[0]
    fv_t = feature_value.T
    emb = _make_emb_kernel(F, V, D, B)
    return emb(fv_t, tables)


# speedup vs baseline: 1.1695x; 1.0043x over previous
import functools

import jax
import jax.numpy as jnp
from jax import lax
from jax.experimental import pallas as pl
from jax.experimental.pallas import tpu as pltpu
from jax.experimental.pallas import tpu_sc as plsc


def _make_emb_kernel(F, V, D, B):
    info = plsc.get_sparse_core_info()
    NC, NS = info.num_cores, info.num_subcores
    NW = NC * NS
    assert B % NW == 0
    BPW = B // NW

    mesh = plsc.VectorSubcoreMesh(core_axis_name="c", subcore_axis_name="s")

    @functools.partial(
        pl.kernel,
        mesh=mesh,
        out_type=jax.ShapeDtypeStruct((F, B, D), jnp.float32),
        scratch_types=[
            pltpu.VMEM((BPW, F), jnp.int32),
            pltpu.VMEM((F, BPW), jnp.int32),
            pltpu.VMEM((2, BPW, D), jnp.float32),
            pltpu.SemaphoreType.DMA,
            pltpu.SemaphoreType.DMA,
            pltpu.SemaphoreType.DMA,
            pltpu.SemaphoreType.DMA,
        ],
        compiler_params=pltpu.CompilerParams(
            use_tc_tiling_on_sc=False, needs_layout_passes=False
        ),
    )
    def emb(fv_hbm, tab_hbm, out_hbm, fv_v, idx_v, rows_v, sg0, sg1, sw0, sw1):
        wid = lax.axis_index("s") * NC + lax.axis_index("c")
        base = wid * BPW
        pltpu.sync_copy(fv_hbm.at[pl.ds(base, BPW)], fv_v)

        def tr_body(j, carry):
            rows = lax.iota(jnp.int32, 16) + j * 16
            for i in range(F):
                col = jnp.full((16,), i, jnp.int32)
                v = plsc.load_gather(fv_v, [rows, col])
                idx_v[i, pl.ds(pl.multiple_of(j * 16, 16), 16)] = v
            return carry

        lax.fori_loop(0, BPW // 16, tr_body, 0)
        sg = (sg0, sg1)
        sw = (sw0, sw1)
        wb = [None, None]
        for i in range(F):
            b = i % 2
            if wb[b] is not None:
                wb[b].wait()
            pltpu.async_copy(tab_hbm.at[i].at[idx_v.at[i]], rows_v.at[b], sg[b]).wait()
            wb[b] = pltpu.async_copy(
                rows_v.at[b], out_hbm.at[i, pl.ds(base, BPW)], sw[b]
            )
        wb[0].wait()
        wb[1].wait()

    return emb


def kernel(feature_value, tables):
    F, V, D = tables.shape
    B = feature_value.shape[0]
    emb = _make_emb_kernel(F, V, D, B)
    return emb(feature_value, tables)


# speedup vs baseline: 1.2913x; 1.1042x over previous
import functools

import jax
import jax.numpy as jnp
from jax import lax
from jax.experimental import pallas as pl
from jax.experimental.pallas import tpu as pltpu
from jax.experimental.pallas import tpu_sc as plsc


def _make_emb_kernel(F, V, D, B):
    info = plsc.get_sparse_core_info()
    NC, NS = info.num_cores, info.num_subcores
    NW = NC * NS
    assert B % NW == 0
    BPW = B // NW

    mesh = plsc.VectorSubcoreMesh(core_axis_name="c", subcore_axis_name="s")

    @functools.partial(
        pl.kernel,
        mesh=mesh,
        out_type=jax.ShapeDtypeStruct((F * B, 128), jnp.float32),
        scratch_types=[
            pltpu.VMEM((BPW, F), jnp.int32),
            pltpu.VMEM((F, BPW), jnp.int32),
            pltpu.VMEM((2, BPW, D), jnp.float32),
            pltpu.SemaphoreType.DMA,
            pltpu.SemaphoreType.DMA,
            pltpu.SemaphoreType.DMA,
            pltpu.SemaphoreType.DMA,
        ],
        compiler_params=pltpu.CompilerParams(
            use_tc_tiling_on_sc=False, needs_layout_passes=False
        ),
    )
    def emb(fv_hbm, tab_hbm, out_hbm, fv_v, idx_v, rows_v, sg0, sg1, sw0, sw1):
        wid = lax.axis_index("s") * NC + lax.axis_index("c")
        base = wid * BPW
        pltpu.sync_copy(fv_hbm.at[pl.ds(base, BPW)], fv_v)

        def tr_body(j, carry):
            rows = lax.iota(jnp.int32, 16) + j * 16
            for i in range(F):
                col = jnp.full((16,), i, jnp.int32)
                v = plsc.load_gather(fv_v, [rows, col])
                idx_v[i, pl.ds(pl.multiple_of(j * 16, 16), 16)] = v
            return carry

        lax.fori_loop(0, BPW // 16, tr_body, 0)
        sg = (sg0, sg1)
        sw = (sw0, sw1)
        wb = [None, None]
        for i in range(F):
            b = i % 2
            if wb[b] is not None:
                wb[b].wait()
            pltpu.async_copy(tab_hbm.at[i].at[idx_v.at[i]], rows_v.at[b], sg[b]).wait()
            wb[b] = pltpu.async_copy(
                rows_v.at[b],
                out_hbm.at[pl.ds(i * B + base, BPW), pl.ds(0, D)],
                sw[b],
            )
        wb[0].wait()
        wb[1].wait()

    return emb


def kernel(feature_value, tables):
    F, V, D = tables.shape
    B = feature_value.shape[0]
    emb = _make_emb_kernel(F, V, D, B)
    s = emb(feature_value, tables)
    return s.reshape(F, B, 128)[:, :, :D]
